# baseline (speedup 1.0000x reference)
import functools

import numpy as np
import jax
import jax.numpy as jnp
from jax import lax
from jax.experimental import pallas as pl
from jax.experimental.pallas import tpu as pltpu
from jax.experimental.pallas import tpu_sc as plsc

_B, _P, _F = 4096, 10, 7
_MIN_SCORE = 0.3

_INFO = plsc.get_sparse_core_info()
_NC, _NS, _L = _INFO.num_cores, _INFO.num_subcores, _INFO.num_lanes
_NW = _NC * _NS
_RW = _B // _NW
_RB = 8
_NBLK = _RW // _RB

_NI = _RB * _P * 3
_NX = _RB * _P * 2
_NCF = _RB * _P


def _build_table():
    def rpc(n_feat):
        j = np.arange(_RB * _P * n_feat, dtype=np.int32)
        return j // (_P * n_feat), (j // n_feat) % _P, j % n_feat

    ib, ip, ic = rpc(3)
    xb, xp, xc = rpc(2)
    cb, cp, _ = rpc(1)
    return np.concatenate([ib, ip, ic, xb, xp, xc, cb, cp])


_TAB_NP = _build_table()
_O_IB, _O_IP, _O_IC = 0, _NI, 2 * _NI
_O_XB, _O_XP, _O_XC = 3 * _NI, 3 * _NI + _NX, 3 * _NI + 2 * _NX
_O_CB, _O_CP = 3 * _NI + 3 * _NX, 3 * _NI + 3 * _NX + _NCF


@functools.partial(
    pl.kernel,
    mesh=plsc.VectorSubcoreMesh(core_axis_name="c", subcore_axis_name="s"),
    out_type=jax.ShapeDtypeStruct((_B, _P, _F), jnp.float32),
    compiler_params=pltpu.CompilerParams(
        needs_layout_passes=False, use_tc_tiling_on_sc=False),
    scratch_types=[
        pltpu.VMEM((_RW, _P, 3), jnp.float32),
        pltpu.VMEM((_RW, _P), jnp.float32),
        pltpu.VMEM((_RW, _P, 2), jnp.float32),
        pltpu.VMEM((_RW, _P, _F), jnp.float32),
        pltpu.VMEM((_TAB_NP.size,), jnp.int32),
    ],
)
def _sc_assemble(idx_hbm, conf_hbm, bbox_hbm, tab_hbm, out_hbm,
                 idx_v, conf_v, bbox_v, out_v, tab):
    wid = lax.axis_index("s") * _NC + lax.axis_index("c")
    rows = pl.ds(wid * _RW, _RW)
    pltpu.sync_copy(idx_hbm.at[rows], idx_v)
    pltpu.sync_copy(conf_hbm.at[rows], conf_v)
    pltpu.sync_copy(bbox_hbm.at[rows], bbox_v)
    pltpu.sync_copy(tab_hbm, tab)

    five = jnp.full((_L,), 5, jnp.int32)
    three = jnp.full((_L,), 3, jnp.int32)
    four = jnp.full((_L,), 4, jnp.int32)

    def blk_step(blk, carry):
        roff = blk * _RB
        for t in range(_NI // _L):
            o = t * _L
            b = tab[pl.ds(_O_IB + o, _L)] + roff
            p = tab[pl.ds(_O_IP + o, _L)]
            c = tab[pl.ds(_O_IC + o, _L)]
            plsc.store_scatter(out_v, [b, p, c],
                               plsc.load_gather(idx_v, [b, p, c]))
        for t in range(_NX // _L):
            o = t * _L
            b = tab[pl.ds(_O_XB + o, _L)] + roff
            p = tab[pl.ds(_O_XP + o, _L)]
            c = tab[pl.ds(_O_XC + o, _L)]
            plsc.store_scatter(out_v, [b, p, c + five],
                               plsc.load_gather(bbox_v, [b, p, c]))
        for t in range(_NCF // _L):
            o = t * _L
            b = tab[pl.ds(_O_CB + o, _L)] + roff
            p = tab[pl.ds(_O_CP + o, _L)]
            cvals = plsc.load_gather(conf_v, [b, p])
            m = jnp.where(cvals > _MIN_SCORE, jnp.float32(0.0),
                          jnp.float32(-1.0))
            plsc.store_scatter(out_v, [b, p, four], cvals)
            plsc.store_scatter(out_v, [b, p, three], m)
        return carry

    lax.fori_loop(0, _NBLK, blk_step, 0)
    pltpu.sync_copy(out_v, out_hbm.at[rows])


def kernel(topk_index, topk_confs, match_bbox_preds, meta):
    del meta
    return _sc_assemble(topk_index, topk_confs, match_bbox_preds,
                        jnp.asarray(_TAB_NP))


# speedup vs baseline: 1.4572x; 1.4572x over previous
import functools

import jax
import jax.numpy as jnp
from jax import lax
from jax.experimental import pallas as pl
from jax.experimental.pallas import tpu as pltpu
from jax.experimental.pallas import tpu_sc as plsc

_B, _P, _F = 4096, 10, 7
_MIN_SCORE = 0.3
_SL, _LN = 16, 128

_INFO = plsc.get_sparse_core_info()
_NC, _NS, _L = _INFO.num_cores, _INFO.num_subcores, _INFO.num_lanes
_NW = _NC * _NS
_RW = _B // _NW
_NREC = _RW * _P


@functools.partial(
    pl.kernel,
    mesh=plsc.VectorSubcoreMesh(core_axis_name="c", subcore_axis_name="s"),
    out_type=jax.ShapeDtypeStruct((_B, _SL, _LN), jnp.float32),
    compiler_params=pltpu.CompilerParams(
        needs_layout_passes=False, use_tc_tiling_on_sc=False),
    scratch_types=[
        pltpu.VMEM((_RW, _P, _L), jnp.float32),
        pltpu.VMEM((_RW, _P, _L), jnp.float32),
        pltpu.VMEM((_RW, _L), jnp.float32),
    ],
)
def _sc_assemble(idx_hbm, conf_hbm, bbox_hbm, out_hbm, out_v, bb_v, cf_v):
    wid = lax.axis_index("s") * _NC + lax.axis_index("c")
    rows = pl.ds(wid * _RW, _RW)
    ppl = pl.ds(0, _P)
    lanes = pl.ds(0, _L)
    pltpu.sync_copy(idx_hbm.at[rows, ppl, lanes], out_v)
    pltpu.sync_copy(bbox_hbm.at[rows, ppl, lanes], bb_v)
    pltpu.sync_copy(conf_hbm.at[rows, lanes], cf_v)

    iota = lax.iota(jnp.int32, _L)
    zero = jnp.zeros((_L,), jnp.int32)
    one = jnp.full((_L,), 1, jnp.int32)
    three = jnp.full((_L,), 3, jnp.int32)
    four = jnp.full((_L,), 4, jnp.int32)
    five = jnp.full((_L,), 5, jnp.int32)
    six = jnp.full((_L,), 6, jnp.int32)

    def grp_step(g, carry):
        j = g * _L + iota
        b = j // _P
        p = j % _P
        plsc.store_scatter(out_v, [b, p, five],
                           plsc.load_gather(bb_v, [b, p, zero]))
        plsc.store_scatter(out_v, [b, p, six],
                           plsc.load_gather(bb_v, [b, p, one]))
        cvals = plsc.load_gather(cf_v, [b, p])
        plsc.store_scatter(out_v, [b, p, four], cvals)
        m = jnp.where(cvals > _MIN_SCORE, jnp.float32(0.0), jnp.float32(-1.0))
        plsc.store_scatter(out_v, [b, p, three], m)
        return carry

    lax.fori_loop(0, _NREC // _L, grp_step, 0)
    pltpu.sync_copy(out_v, out_hbm.at[rows, ppl, lanes])


def kernel(topk_index, topk_confs, match_bbox_preds, meta):
    del meta
    idxp = jnp.pad(topk_index, ((0, 0), (0, _SL - _P), (0, _LN - 3)))
    confp = jnp.pad(topk_confs, ((0, 0), (0, _LN - _P)))
    bboxp = jnp.pad(match_bbox_preds, ((0, 0), (0, _SL - _P), (0, _LN - 2)))
    outp = _sc_assemble(idxp, confp, bboxp)
    return outp[:, :_P, :_F]
